# baseline (speedup 1.0000x reference)
import functools

import jax
import jax.numpy as jnp
from jax import lax
from jax.experimental import pallas as pl
from jax.experimental.pallas import tpu as pltpu
from jax.experimental.pallas import tpu_sc as plsc

_SPARSITY_RATIO = 0.5
_MIN_TOKENS = 16
_MAX_TOKENS = 4096

_NC = 2
_NS = 16
_NW = _NC * _NS


def _norms_body(x_ref, o_ref):
    x = x_ref[...]
    sq = x * x
    D = x.shape[-1]
    acc = sq[:, :, 0:128]
    for c in range(1, D // 128):
        acc = acc + sq[:, :, c * 128:(c + 1) * 128]
    h = acc[:, :, 0:8]
    for t in range(1, 16):
        h = h + acc[:, :, t * 8:t * 8 + 8]
    b1 = h[:, :, 0:4] + h[:, :, 4:8]
    b2 = b1[:, :, 0:2] + b1[:, :, 2:4]
    s = b2[:, :, 0:1] + b2[:, :, 1:2]
    o_ref[...] = jnp.sqrt(s)


def _topk_body(nrow_ref, ncol_ref, idx_ref, gidx_ref, rank_ref, *, S, K):
    b = pl.program_id(0)
    n_row = nrow_ref[...].reshape(1, S)
    CI = 256
    for ic in range(S // CI):
        n_col = ncol_ref[0, pl.ds(ic * CI, CI), :]
        i_iota = ic * CI + lax.broadcasted_iota(jnp.int32, (CI, S), 0)
        j_iota = lax.broadcasted_iota(jnp.int32, (CI, S), 1)
        gt = n_row > n_col
        eq = n_row == n_col
        cell = jnp.logical_or(gt, jnp.logical_and(eq, j_iota < i_iota))
        rank_ref[pl.ds(ic * CI, CI), :] = jnp.sum(
            jnp.where(cell, 1.0, 0.0), axis=1, keepdims=True)
    acc = jnp.zeros((1, K), jnp.float32)
    CR = 512
    for rc in range(S // CR):
        rank_chunk = rank_ref[pl.ds(rc * CR, CR), :]
        r_iota = lax.broadcasted_iota(jnp.int32, (CR, K), 1).astype(jnp.float32)
        i_val = (rc * CR + lax.broadcasted_iota(jnp.int32, (CR, K), 0)
                 ).astype(jnp.float32)
        acc = acc + jnp.sum(
            jnp.where(rank_chunk == r_iota, i_val, 0.0), axis=0, keepdims=True)
    idx = acc.astype(jnp.int32)
    idx_ref[...] = idx[:, None, :]
    gidx_ref[...] = (idx + b * S)[:, None, :]


def _make_sc_gather(BK, D, bpw, CH):
    NCH = bpw // CH
    mesh = plsc.VectorSubcoreMesh(
        core_axis_name="c", subcore_axis_name="s",
        num_cores=_NC, num_subcores=_NS)

    @functools.partial(
        pl.kernel,
        out_type=jax.ShapeDtypeStruct((BK, D), jnp.float32),
        mesh=mesh,
        scratch_types=[
            pltpu.VMEM((NCH, CH), jnp.int32),
            pltpu.VMEM((CH, D), jnp.float32),
            pltpu.VMEM((CH, D), jnp.float32),
            pltpu.SemaphoreType.DMA,
            pltpu.SemaphoreType.DMA,
            pltpu.SemaphoreType.DMA,
            pltpu.SemaphoreType.DMA,
        ],
    )
    def gather_k(table_hbm, gidx_hbm, out_hbm, idx_v, buf0, buf1,
                 gsem0, gsem1, osem0, osem1):
        wid = lax.axis_index("s") * _NC + lax.axis_index("c")
        base = wid * bpw
        pltpu.sync_copy(gidx_hbm.at[wid], idx_v)
        bufs = (buf0, buf1)
        gsems = (gsem0, gsem1)
        osems = (osem0, osem1)
        g = [None] * NCH
        o = [None] * NCH
        g[0] = pltpu.make_async_copy(table_hbm.at[idx_v.at[0]], bufs[0], gsems[0])
        g[0].start()
        for c in range(NCH):
            cur = c & 1
            g[c].wait()
            if c + 1 < NCH:
                if c >= 1:
                    o[c - 1].wait()
                g[c + 1] = pltpu.make_async_copy(
                    table_hbm.at[idx_v.at[c + 1]], bufs[1 - cur], gsems[1 - cur])
                g[c + 1].start()
            o[c] = pltpu.make_async_copy(
                bufs[cur], out_hbm.at[pl.ds(base + c * CH, CH)], osems[cur])
            o[c].start()
        if NCH >= 2:
            o[NCH - 2].wait()
        o[NCH - 1].wait()

    return gather_k


def kernel(tokens):
    B, S, D = tokens.shape
    target = max(_MIN_TOKENS, min(int(S * _SPARSITY_RATIO), _MAX_TOKENS))
    K = min(target, S)

    BS = 256
    ncol = pl.pallas_call(
        _norms_body,
        grid=(B, S // BS),
        in_specs=[pl.BlockSpec((1, BS, D), lambda b, s: (b, s, 0))],
        out_specs=pl.BlockSpec((1, BS, 1), lambda b, s: (b, s, 0)),
        out_shape=jax.ShapeDtypeStruct((B, S, 1), jnp.float32),
    )(tokens)

    norms3 = ncol.reshape(B, 1, S)
    idx3, gidx3 = pl.pallas_call(
        functools.partial(_topk_body, S=S, K=K),
        grid=(B,),
        in_specs=[
            pl.BlockSpec((1, 1, S), lambda b: (b, 0, 0)),
            pl.BlockSpec((1, S, 1), lambda b: (b, 0, 0)),
        ],
        out_specs=[
            pl.BlockSpec((1, 1, K), lambda b: (b, 0, 0)),
            pl.BlockSpec((1, 1, K), lambda b: (b, 0, 0)),
        ],
        out_shape=[
            jax.ShapeDtypeStruct((B, 1, K), jnp.int32),
            jax.ShapeDtypeStruct((B, 1, K), jnp.int32),
        ],
        scratch_shapes=[pltpu.VMEM((S, 1), jnp.float32)],
    )(norms3, ncol)

    idx = idx3.reshape(B, K)
    BK = B * K
    bpw = BK // _NW
    CH = 8
    gidx = gidx3.reshape(_NW, bpw // CH, CH)
    table = tokens.reshape(B * S, D)
    out = _make_sc_gather(BK, D, bpw, CH)(table, gidx)
    return out.reshape(B, K, D), idx


# speedup vs baseline: 1.1481x; 1.1481x over previous
import functools

import jax
import jax.numpy as jnp
from jax import lax
from jax.experimental import pallas as pl
from jax.experimental.pallas import tpu as pltpu
from jax.experimental.pallas import tpu_sc as plsc

_SPARSITY_RATIO = 0.5
_MIN_TOKENS = 16
_MAX_TOKENS = 4096

_NC = 2
_NS = 16
_NW = _NC * _NS


def _fused_body(x_ref, idx_ref, gidx_ref, nscr, rscr, *, S, K, BS, D):
    b = pl.program_id(0)
    s = pl.program_id(1)
    last = S // BS

    @pl.when(s < last)
    def _norm_step():
        x = x_ref[...].reshape(BS, D)
        sq = x * x
        for r in range(BS // 128):
            acc = sq[r * 128:(r + 1) * 128, 0:128]
            for c in range(1, D // 128):
                acc = acc + sq[r * 128:(r + 1) * 128, c * 128:(c + 1) * 128]
            ta = jnp.transpose(acc)
            h = ta[0:8, :]
            for t in range(1, 16):
                h = h + ta[t * 8:t * 8 + 8, :]
            b1 = h[0:4, :] + h[4:8, :]
            b2 = b1[0:2, :] + b1[2:4, :]
            sm = b2[0:1, :] + b2[1:2, :]
            nscr[:, pl.ds(s * BS + r * 128, 128)] = jnp.sqrt(sm)

    @pl.when(s == last)
    def _topk_step():
        nrow = nscr[...]
        CI = 256
        for ic in range(S // CI):
            n_col = jnp.transpose(nrow[:, ic * CI:(ic + 1) * CI])
            i_iota = ic * CI + lax.broadcasted_iota(jnp.int32, (CI, S), 0)
            j_iota = lax.broadcasted_iota(jnp.int32, (CI, S), 1)
            gt = nrow > n_col
            eq = nrow == n_col
            cell = jnp.logical_or(gt, jnp.logical_and(eq, j_iota < i_iota))
            rscr[pl.ds(ic * CI, CI), :] = jnp.sum(
                jnp.where(cell, 1.0, 0.0), axis=1, keepdims=True)
        acc = jnp.zeros((1, K), jnp.float32)
        CR = 512
        for rc in range(S // CR):
            rank_chunk = rscr[pl.ds(rc * CR, CR), :]
            r_iota = lax.broadcasted_iota(jnp.int32, (CR, K), 1).astype(
                jnp.float32)
            i_val = (rc * CR + lax.broadcasted_iota(jnp.int32, (CR, K), 0)
                     ).astype(jnp.float32)
            acc = acc + jnp.sum(
                jnp.where(rank_chunk == r_iota, i_val, 0.0),
                axis=0, keepdims=True)
        idx = acc.astype(jnp.int32)
        idx_ref[...] = idx[:, None, :]
        gidx_ref[...] = (idx + b * S)[:, None, :]


def _make_sc_gather(BK, D, bpw, CH):
    NCH = bpw // CH
    mesh = plsc.VectorSubcoreMesh(
        core_axis_name="c", subcore_axis_name="s",
        num_cores=_NC, num_subcores=_NS)

    @functools.partial(
        pl.kernel,
        out_type=jax.ShapeDtypeStruct((BK, D), jnp.float32),
        mesh=mesh,
        scratch_types=[
            pltpu.VMEM((NCH, CH), jnp.int32),
            pltpu.VMEM((CH, D), jnp.float32),
            pltpu.VMEM((CH, D), jnp.float32),
            pltpu.SemaphoreType.DMA,
            pltpu.SemaphoreType.DMA,
            pltpu.SemaphoreType.DMA,
            pltpu.SemaphoreType.DMA,
        ],
    )
    def gather_k(table_hbm, gidx_hbm, out_hbm, idx_v, buf0, buf1,
                 gsem0, gsem1, osem0, osem1):
        wid = lax.axis_index("s") * _NC + lax.axis_index("c")
        base = wid * bpw
        pltpu.sync_copy(gidx_hbm.at[wid], idx_v)
        bufs = (buf0, buf1)
        gsems = (gsem0, gsem1)
        osems = (osem0, osem1)
        g = [None] * NCH
        o = [None] * NCH
        g[0] = pltpu.make_async_copy(table_hbm.at[idx_v.at[0]], bufs[0], gsems[0])
        g[0].start()
        for c in range(NCH):
            cur = c & 1
            g[c].wait()
            if c + 1 < NCH:
                if c >= 1:
                    o[c - 1].wait()
                g[c + 1] = pltpu.make_async_copy(
                    table_hbm.at[idx_v.at[c + 1]], bufs[1 - cur], gsems[1 - cur])
                g[c + 1].start()
            o[c] = pltpu.make_async_copy(
                bufs[cur], out_hbm.at[pl.ds(base + c * CH, CH)], osems[cur])
            o[c].start()
        if NCH >= 2:
            o[NCH - 2].wait()
        o[NCH - 1].wait()

    return gather_k


def kernel(tokens):
    B, S, D = tokens.shape
    target = max(_MIN_TOKENS, min(int(S * _SPARSITY_RATIO), _MAX_TOKENS))
    K = min(target, S)

    BS = 256
    last = S // BS
    idx3, gidx3 = pl.pallas_call(
        functools.partial(_fused_body, S=S, K=K, BS=BS, D=D),
        grid=(B, last + 1),
        in_specs=[pl.BlockSpec(
            (1, BS, D), lambda b, s: (b, jnp.minimum(s, last - 1), 0))],
        out_specs=[
            pl.BlockSpec((1, 1, K), lambda b, s: (b, 0, 0)),
            pl.BlockSpec((1, 1, K), lambda b, s: (b, 0, 0)),
        ],
        out_shape=[
            jax.ShapeDtypeStruct((B, 1, K), jnp.int32),
            jax.ShapeDtypeStruct((B, 1, K), jnp.int32),
        ],
        scratch_shapes=[
            pltpu.VMEM((1, S), jnp.float32),
            pltpu.VMEM((S, 1), jnp.float32),
        ],
    )(tokens)

    idx = idx3.reshape(B, K)
    BK = B * K
    bpw = BK // _NW
    CH = 8
    gidx = gidx3.reshape(_NW, bpw // CH, CH)
    table = tokens.reshape(B * S, D)
    out = _make_sc_gather(BK, D, bpw, CH)(table, gidx)
    return out.reshape(B, K, D), idx


# speedup vs baseline: 1.1719x; 1.0207x over previous
import functools

import jax
import jax.numpy as jnp
from jax import lax
from jax.experimental import pallas as pl
from jax.experimental.pallas import tpu as pltpu
from jax.experimental.pallas import tpu_sc as plsc

_SPARSITY_RATIO = 0.5
_MIN_TOKENS = 16
_MAX_TOKENS = 4096

_NC = 2
_NS = 16
_NW = _NC * _NS


def _fused_body(x_ref, idx_ref, gidx_ref, nscr, rscr, *, S, K, BS, D):
    b = pl.program_id(0)
    s = pl.program_id(1)
    last = S // BS

    @pl.when(s < last)
    def _norm_step():
        x = x_ref[...].reshape(BS, D)
        sq = x * x
        for r in range(BS // 128):
            acc = sq[r * 128:(r + 1) * 128, 0:128]
            for c in range(1, D // 128):
                acc = acc + sq[r * 128:(r + 1) * 128, c * 128:(c + 1) * 128]
            ta = jnp.transpose(acc)
            h = ta[0:8, :]
            for t in range(1, 16):
                h = h + ta[t * 8:t * 8 + 8, :]
            b1 = h[0:4, :] + h[4:8, :]
            b2 = b1[0:2, :] + b1[2:4, :]
            sm = b2[0:1, :] + b2[1:2, :]
            nscr[:, pl.ds(s * BS + r * 128, 128)] = jnp.sqrt(sm)

    @pl.when(s == last)
    def _topk_step():
        nrow = nscr[...]
        CI = 256
        for ic in range(S // CI):
            n_col = jnp.transpose(nrow[:, ic * CI:(ic + 1) * CI])
            i_iota = ic * CI + lax.broadcasted_iota(jnp.int32, (CI, S), 0)
            j_iota = lax.broadcasted_iota(jnp.int32, (CI, S), 1)
            gt = nrow > n_col
            eq = nrow == n_col
            cell = jnp.logical_or(gt, jnp.logical_and(eq, j_iota < i_iota))
            rscr[pl.ds(ic * CI, CI), :] = jnp.sum(
                jnp.where(cell, 1.0, 0.0), axis=1, keepdims=True)
        acc = jnp.zeros((1, K), jnp.float32)
        CR = 512
        for rc in range(S // CR):
            rank_chunk = rscr[pl.ds(rc * CR, CR), :]
            r_iota = lax.broadcasted_iota(jnp.int32, (CR, K), 1).astype(
                jnp.float32)
            i_val = (rc * CR + lax.broadcasted_iota(jnp.int32, (CR, K), 0)
                     ).astype(jnp.float32)
            acc = acc + jnp.sum(
                jnp.where(rank_chunk == r_iota, i_val, 0.0),
                axis=0, keepdims=True)
        idx = acc.astype(jnp.int32)
        idx_ref[...] = idx[:, None, :]
        gidx_ref[...] = (idx + b * S)[:, None, :]


def _make_sc_gather(BK, D, bpw, CH):
    NCH = bpw // CH
    mesh = plsc.VectorSubcoreMesh(
        core_axis_name="c", subcore_axis_name="s",
        num_cores=_NC, num_subcores=_NS)

    @functools.partial(
        pl.kernel,
        out_type=jax.ShapeDtypeStruct((BK, D), jnp.float32),
        mesh=mesh,
        scratch_types=[
            pltpu.VMEM((NCH, CH), jnp.int32),
            pltpu.VMEM((CH, D), jnp.float32),
            pltpu.VMEM((CH, D), jnp.float32),
            pltpu.VMEM((CH, D), jnp.float32),
            pltpu.SemaphoreType.DMA,
            pltpu.SemaphoreType.DMA,
            pltpu.SemaphoreType.DMA,
            pltpu.SemaphoreType.DMA,
            pltpu.SemaphoreType.DMA,
            pltpu.SemaphoreType.DMA,
        ],
    )
    def gather_k(table_hbm, gidx_hbm, out_hbm, idx_v, buf0, buf1, buf2,
                 gsem0, gsem1, gsem2, osem0, osem1, osem2):
        wid = lax.axis_index("s") * _NC + lax.axis_index("c")
        base = wid * bpw
        pltpu.sync_copy(gidx_hbm.at[wid], idx_v)
        bufs = (buf0, buf1, buf2)
        gsems = (gsem0, gsem1, gsem2)
        osems = (osem0, osem1, osem2)

        def gmake(c):
            return pltpu.make_async_copy(
                table_hbm.at[idx_v.at[c]], bufs[c % 3], gsems[c % 3])

        def omake(c):
            return pltpu.make_async_copy(
                bufs[c % 3], out_hbm.at[pl.ds(base + c * CH, CH)],
                osems[c % 3])

        g = [None] * NCH
        o = [None] * NCH
        g[0] = gmake(0)
        g[0].start()
        if NCH > 1:
            g[1] = gmake(1)
            g[1].start()
        for c in range(NCH):
            g[c].wait()
            o[c] = omake(c)
            o[c].start()
            if c + 2 < NCH:
                if c >= 1:
                    o[c - 1].wait()
                g[c + 2] = gmake(c + 2)
                g[c + 2].start()
        for c in range(max(0, NCH - 3), NCH):
            o[c].wait()

    return gather_k


def kernel(tokens):
    B, S, D = tokens.shape
    target = max(_MIN_TOKENS, min(int(S * _SPARSITY_RATIO), _MAX_TOKENS))
    K = min(target, S)

    BS = 256
    last = S // BS
    idx3, gidx3 = pl.pallas_call(
        functools.partial(_fused_body, S=S, K=K, BS=BS, D=D),
        grid=(B, last + 1),
        in_specs=[pl.BlockSpec(
            (1, BS, D), lambda b, s: (b, jnp.minimum(s, last - 1), 0))],
        out_specs=[
            pl.BlockSpec((1, 1, K), lambda b, s: (b, 0, 0)),
            pl.BlockSpec((1, 1, K), lambda b, s: (b, 0, 0)),
        ],
        out_shape=[
            jax.ShapeDtypeStruct((B, 1, K), jnp.int32),
            jax.ShapeDtypeStruct((B, 1, K), jnp.int32),
        ],
        scratch_shapes=[
            pltpu.VMEM((1, S), jnp.float32),
            pltpu.VMEM((S, 1), jnp.float32),
        ],
    )(tokens)

    idx = idx3.reshape(B, K)
    BK = B * K
    bpw = BK // _NW
    CH = 8
    gidx = gidx3.reshape(_NW, bpw // CH, CH)
    table = tokens.reshape(B * S, D)
    out = _make_sc_gather(BK, D, bpw, CH)(table, gidx)
    return out.reshape(B, K, D), idx


# speedup vs baseline: 1.1765x; 1.0039x over previous
import functools

import jax
import jax.numpy as jnp
from jax import lax
from jax.experimental import pallas as pl
from jax.experimental.pallas import tpu as pltpu
from jax.experimental.pallas import tpu_sc as plsc

_SPARSITY_RATIO = 0.5
_MIN_TOKENS = 16
_MAX_TOKENS = 4096

_NC = 2
_NS = 16
_NW = _NC * _NS


def _fused_body(x_ref, idx_ref, gidx_ref, nscr, rscr, *, S, K, BS, D):
    b = pl.program_id(0)
    s = pl.program_id(1)
    last = S // BS

    x = x_ref[...].reshape(BS, D)
    sq = x * x
    for r in range(BS // 128):
        acc = sq[r * 128:(r + 1) * 128, 0:128]
        for c in range(1, D // 128):
            acc = acc + sq[r * 128:(r + 1) * 128, c * 128:(c + 1) * 128]
        ta = jnp.transpose(acc)
        h = ta[0:8, :]
        for t in range(1, 16):
            h = h + ta[t * 8:t * 8 + 8, :]
        b1 = h[0:4, :] + h[4:8, :]
        b2 = b1[0:2, :] + b1[2:4, :]
        sm = b2[0:1, :] + b2[1:2, :]
        nscr[:, pl.ds(s * BS + r * 128, 128)] = jnp.sqrt(sm)

    @pl.when(s == last - 1)
    def _topk_step():
        nrow = nscr[...]
        CI = 256
        for ic in range(S // CI):
            n_col = jnp.transpose(nrow[:, ic * CI:(ic + 1) * CI])
            i_iota = ic * CI + lax.broadcasted_iota(jnp.int32, (CI, S), 0)
            j_iota = lax.broadcasted_iota(jnp.int32, (CI, S), 1)
            gt = nrow > n_col
            eq = nrow == n_col
            cell = jnp.logical_or(gt, jnp.logical_and(eq, j_iota < i_iota))
            rscr[pl.ds(ic * CI, CI), :] = jnp.sum(
                jnp.where(cell, 1.0, 0.0), axis=1, keepdims=True)
        acc = jnp.zeros((1, K), jnp.float32)
        CR = 512
        for rc in range(S // CR):
            rank_chunk = rscr[pl.ds(rc * CR, CR), :]
            r_iota = lax.broadcasted_iota(jnp.int32, (CR, K), 1).astype(
                jnp.float32)
            i_val = (rc * CR + lax.broadcasted_iota(jnp.int32, (CR, K), 0)
                     ).astype(jnp.float32)
            acc = acc + jnp.sum(
                jnp.where(rank_chunk == r_iota, i_val, 0.0),
                axis=0, keepdims=True)
        idx = acc.astype(jnp.int32)
        idx_ref[...] = idx[:, None, :]
        gidx_ref[...] = (idx + b * S)[:, None, :]


def _make_sc_gather(BK, D, bpw, CH):
    NCH = bpw // CH
    mesh = plsc.VectorSubcoreMesh(
        core_axis_name="c", subcore_axis_name="s",
        num_cores=_NC, num_subcores=_NS)

    @functools.partial(
        pl.kernel,
        out_type=jax.ShapeDtypeStruct((BK, D), jnp.float32),
        mesh=mesh,
        scratch_types=[
            pltpu.VMEM((NCH, CH), jnp.int32),
            pltpu.VMEM((CH, D), jnp.float32),
            pltpu.VMEM((CH, D), jnp.float32),
            pltpu.VMEM((CH, D), jnp.float32),
            pltpu.SemaphoreType.DMA,
            pltpu.SemaphoreType.DMA,
            pltpu.SemaphoreType.DMA,
            pltpu.SemaphoreType.DMA,
            pltpu.SemaphoreType.DMA,
            pltpu.SemaphoreType.DMA,
        ],
    )
    def gather_k(table_hbm, gidx_hbm, out_hbm, idx_v, buf0, buf1, buf2,
                 gsem0, gsem1, gsem2, osem0, osem1, osem2):
        wid = lax.axis_index("s") * _NC + lax.axis_index("c")
        base = wid * bpw
        pltpu.sync_copy(gidx_hbm.at[wid], idx_v)
        bufs = (buf0, buf1, buf2)
        gsems = (gsem0, gsem1, gsem2)
        osems = (osem0, osem1, osem2)

        def gmake(c):
            return pltpu.make_async_copy(
                table_hbm.at[idx_v.at[c]], bufs[c % 3], gsems[c % 3])

        def omake(c):
            return pltpu.make_async_copy(
                bufs[c % 3], out_hbm.at[pl.ds(base + c * CH, CH)],
                osems[c % 3])

        g = [None] * NCH
        o = [None] * NCH
        g[0] = gmake(0)
        g[0].start()
        if NCH > 1:
            g[1] = gmake(1)
            g[1].start()
        for c in range(NCH):
            g[c].wait()
            o[c] = omake(c)
            o[c].start()
            if c + 2 < NCH:
                if c >= 1:
                    o[c - 1].wait()
                g[c + 2] = gmake(c + 2)
                g[c + 2].start()
        for c in range(max(0, NCH - 3), NCH):
            o[c].wait()

    return gather_k


def kernel(tokens):
    B, S, D = tokens.shape
    target = max(_MIN_TOKENS, min(int(S * _SPARSITY_RATIO), _MAX_TOKENS))
    K = min(target, S)

    BS = 256
    last = S // BS
    idx3, gidx3 = pl.pallas_call(
        functools.partial(_fused_body, S=S, K=K, BS=BS, D=D),
        grid=(B, last),
        in_specs=[pl.BlockSpec((1, BS, D), lambda b, s: (b, s, 0))],
        out_specs=[
            pl.BlockSpec((1, 1, K), lambda b, s: (b, 0, 0)),
            pl.BlockSpec((1, 1, K), lambda b, s: (b, 0, 0)),
        ],
        out_shape=[
            jax.ShapeDtypeStruct((B, 1, K), jnp.int32),
            jax.ShapeDtypeStruct((B, 1, K), jnp.int32),
        ],
        scratch_shapes=[
            pltpu.VMEM((1, S), jnp.float32),
            pltpu.VMEM((S, 1), jnp.float32),
        ],
    )(tokens)

    idx = idx3.reshape(B, K)
    BK = B * K
    bpw = BK // _NW
    CH = 8
    gidx = gidx3.reshape(_NW, bpw // CH, CH)
    table = tokens.reshape(B * S, D)
    out = _make_sc_gather(BK, D, bpw, CH)(table, gidx)
    return out.reshape(B, K, D), idx


# speedup vs baseline: 1.2436x; 1.0571x over previous
import functools

import jax
import jax.numpy as jnp
from jax import lax
from jax.experimental import pallas as pl
from jax.experimental.pallas import tpu as pltpu
from jax.experimental.pallas import tpu_sc as plsc

_SPARSITY_RATIO = 0.5
_MIN_TOKENS = 16
_MAX_TOKENS = 4096

_NC = 2
_NS = 16
_NW = _NC * _NS


def _fused_body(x_ref, idx_ref, gidx_ref, nscr, rscr, *, S, K, BS, D):
    b = pl.program_id(0)
    s = pl.program_id(1)
    last = S // BS

    x = x_ref[...].reshape(BS, D)
    sq = x * x
    for r in range(BS // 128):
        acc = sq[r * 128:(r + 1) * 128, 0:128]
        for c in range(1, D // 128):
            acc = acc + sq[r * 128:(r + 1) * 128, c * 128:(c + 1) * 128]
        ta = jnp.transpose(acc)
        h = ta[0:8, :]
        for t in range(1, 16):
            h = h + ta[t * 8:t * 8 + 8, :]
        b1 = h[0:4, :] + h[4:8, :]
        b2 = b1[0:2, :] + b1[2:4, :]
        sm = b2[0:1, :] + b2[1:2, :]
        nscr[:, pl.ds(s * BS + r * 128, 128)] = jnp.sqrt(sm)

    @pl.when(s == last - 1)
    def _topk_step():
        nrow = nscr[...]
        CI = 256
        for ic in range(S // CI):
            n_col = jnp.transpose(nrow[:, ic * CI:(ic + 1) * CI])
            i_iota = ic * CI + lax.broadcasted_iota(jnp.int32, (CI, S), 0)
            j_iota = lax.broadcasted_iota(jnp.int32, (CI, S), 1)
            gt = nrow > n_col
            eq = nrow == n_col
            cell = jnp.logical_or(gt, jnp.logical_and(eq, j_iota < i_iota))
            rscr[pl.ds(ic * CI, CI), :] = jnp.sum(
                jnp.where(cell, 1.0, 0.0), axis=1, keepdims=True)
        acc = jnp.zeros((1, K), jnp.float32)
        CR = 512
        for rc in range(S // CR):
            rank_chunk = rscr[pl.ds(rc * CR, CR), :]
            r_iota = lax.broadcasted_iota(jnp.int32, (CR, K), 1).astype(
                jnp.float32)
            i_val = (rc * CR + lax.broadcasted_iota(jnp.int32, (CR, K), 0)
                     ).astype(jnp.float32)
            acc = acc + jnp.sum(
                jnp.where(rank_chunk == r_iota, i_val, 0.0),
                axis=0, keepdims=True)
        idx = acc.astype(jnp.int32)
        idx_ref[...] = idx[:, None, :]
        gidx_ref[...] = (idx + b * S)[:, None, :]


def _make_sc_gather(BK, D, bpw, CH):
    NCH = bpw // CH
    mesh = plsc.VectorSubcoreMesh(
        core_axis_name="c", subcore_axis_name="s",
        num_cores=_NC, num_subcores=_NS)

    @functools.partial(
        pl.kernel,
        out_type=jax.ShapeDtypeStruct((BK, D), jnp.float32),
        mesh=mesh,
        scratch_types=[
            pltpu.VMEM((NCH, CH), jnp.int32),
            pltpu.VMEM((CH, D), jnp.float32),
            pltpu.VMEM((CH, D), jnp.float32),
            pltpu.VMEM((CH, D), jnp.float32),
            pltpu.SemaphoreType.DMA,
            pltpu.SemaphoreType.DMA,
            pltpu.SemaphoreType.DMA,
            pltpu.SemaphoreType.DMA,
            pltpu.SemaphoreType.DMA,
            pltpu.SemaphoreType.DMA,
        ],
    )
    def gather_k(table_hbm, gidx_hbm, out_hbm, idx_v, buf0, buf1, buf2,
                 gsem0, gsem1, gsem2, osem0, osem1, osem2):
        wid = lax.axis_index("s") * _NC + lax.axis_index("c")
        base = wid * bpw
        pltpu.sync_copy(gidx_hbm.at[wid], idx_v)
        bufs = (buf0, buf1, buf2)
        gsems = (gsem0, gsem1, gsem2)
        osems = (osem0, osem1, osem2)

        def gmake(c):
            return pltpu.make_async_copy(
                table_hbm.at[idx_v.at[c]], bufs[c % 3], gsems[c % 3])

        def omake(c):
            return pltpu.make_async_copy(
                bufs[c % 3], out_hbm.at[pl.ds(base + c * CH, CH)],
                osems[c % 3])

        g = [None] * NCH
        o = [None] * NCH
        g[0] = gmake(0)
        g[0].start()
        if NCH > 1:
            g[1] = gmake(1)
            g[1].start()
        for c in range(NCH):
            g[c].wait()
            o[c] = omake(c)
            o[c].start()
            if c + 2 < NCH:
                if c >= 1:
                    o[c - 1].wait()
                g[c + 2] = gmake(c + 2)
                g[c + 2].start()
        for c in range(max(0, NCH - 3), NCH):
            o[c].wait()

    return gather_k


def kernel(tokens):
    B, S, D = tokens.shape
    target = max(_MIN_TOKENS, min(int(S * _SPARSITY_RATIO), _MAX_TOKENS))
    K = min(target, S)

    BS = 512
    last = S // BS
    idx3, gidx3 = pl.pallas_call(
        functools.partial(_fused_body, S=S, K=K, BS=BS, D=D),
        grid=(B, last),
        in_specs=[pl.BlockSpec((1, BS, D), lambda b, s: (b, s, 0))],
        out_specs=[
            pl.BlockSpec((1, 1, K), lambda b, s: (b, 0, 0)),
            pl.BlockSpec((1, 1, K), lambda b, s: (b, 0, 0)),
        ],
        out_shape=[
            jax.ShapeDtypeStruct((B, 1, K), jnp.int32),
            jax.ShapeDtypeStruct((B, 1, K), jnp.int32),
        ],
        scratch_shapes=[
            pltpu.VMEM((1, S), jnp.float32),
            pltpu.VMEM((S, 1), jnp.float32),
        ],
    )(tokens)

    idx = idx3.reshape(B, K)
    BK = B * K
    bpw = BK // _NW
    CH = 8
    gidx = gidx3.reshape(_NW, bpw // CH, CH)
    table = tokens.reshape(B * S, D)
    out = _make_sc_gather(BK, D, bpw, CH)(table, gidx)
    return out.reshape(B, K, D), idx


# speedup vs baseline: 1.3135x; 1.0562x over previous
import functools

import jax
import jax.numpy as jnp
from jax import lax
from jax.experimental import pallas as pl
from jax.experimental.pallas import tpu as pltpu
from jax.experimental.pallas import tpu_sc as plsc

_SPARSITY_RATIO = 0.5
_MIN_TOKENS = 16
_MAX_TOKENS = 4096

_NC = 2
_NS = 16
_NW = _NC * _NS


def _fused_body(x_ref, idx_ref, gidx_ref, nscr, rscr, *, S, K, BS, D):
    b = pl.program_id(0)
    s = pl.program_id(1)
    last = S // BS

    x = x_ref[...].reshape(BS, D)
    sq = x * x
    for r in range(BS // 128):
        acc = sq[r * 128:(r + 1) * 128, 0:128]
        for c in range(1, D // 128):
            acc = acc + sq[r * 128:(r + 1) * 128, c * 128:(c + 1) * 128]
        ta = jnp.transpose(acc)
        h = ta[0:8, :]
        for t in range(1, 16):
            h = h + ta[t * 8:t * 8 + 8, :]
        b1 = h[0:4, :] + h[4:8, :]
        b2 = b1[0:2, :] + b1[2:4, :]
        sm = b2[0:1, :] + b2[1:2, :]
        nscr[:, pl.ds(s * BS + r * 128, 128)] = jnp.sqrt(sm)

    @pl.when(s == last - 1)
    def _topk_step():
        nrow = nscr[...]
        CI = 256
        for ic in range(S // CI):
            n_col = jnp.transpose(nrow[:, ic * CI:(ic + 1) * CI])
            i_iota = ic * CI + lax.broadcasted_iota(jnp.int32, (CI, S), 0)
            j_iota = lax.broadcasted_iota(jnp.int32, (CI, S), 1)
            gt = nrow > n_col
            eq = nrow == n_col
            cell = jnp.logical_or(gt, jnp.logical_and(eq, j_iota < i_iota))
            rscr[pl.ds(ic * CI, CI), :] = jnp.sum(
                jnp.where(cell, 1.0, 0.0), axis=1, keepdims=True)
        acc = jnp.zeros((1, K), jnp.float32)
        CR = 512
        for rc in range(S // CR):
            rank_chunk = rscr[pl.ds(rc * CR, CR), :]
            r_iota = lax.broadcasted_iota(jnp.int32, (CR, K), 1).astype(
                jnp.float32)
            i_val = (rc * CR + lax.broadcasted_iota(jnp.int32, (CR, K), 0)
                     ).astype(jnp.float32)
            acc = acc + jnp.sum(
                jnp.where(rank_chunk == r_iota, i_val, 0.0),
                axis=0, keepdims=True)
        idx = acc.astype(jnp.int32)
        idx_ref[...] = idx[:, None, :]
        gidx_ref[...] = (idx + b * S)[:, None, :]


def _make_sc_gather(BK, D, bpw, CH):
    NCH = bpw // CH
    mesh = plsc.VectorSubcoreMesh(
        core_axis_name="c", subcore_axis_name="s",
        num_cores=_NC, num_subcores=_NS)

    @functools.partial(
        pl.kernel,
        out_type=jax.ShapeDtypeStruct((BK, D), jnp.float32),
        mesh=mesh,
        scratch_types=[
            pltpu.VMEM((NCH, CH), jnp.int32),
            pltpu.VMEM((CH, D), jnp.float32),
            pltpu.VMEM((CH, D), jnp.float32),
            pltpu.VMEM((CH, D), jnp.float32),
            pltpu.SemaphoreType.DMA,
            pltpu.SemaphoreType.DMA,
            pltpu.SemaphoreType.DMA,
            pltpu.SemaphoreType.DMA,
            pltpu.SemaphoreType.DMA,
            pltpu.SemaphoreType.DMA,
        ],
    )
    def gather_k(table_hbm, gidx_hbm, out_hbm, idx_v, buf0, buf1, buf2,
                 gsem0, gsem1, gsem2, osem0, osem1, osem2):
        wid = lax.axis_index("s") * _NC + lax.axis_index("c")
        base = wid * bpw
        pltpu.sync_copy(gidx_hbm.at[wid], idx_v)
        bufs = (buf0, buf1, buf2)
        gsems = (gsem0, gsem1, gsem2)
        osems = (osem0, osem1, osem2)

        def gmake(c):
            return pltpu.make_async_copy(
                table_hbm.at[idx_v.at[c]], bufs[c % 3], gsems[c % 3])

        def omake(c):
            return pltpu.make_async_copy(
                bufs[c % 3], out_hbm.at[pl.ds(base + c * CH, CH)],
                osems[c % 3])

        g = [None] * NCH
        o = [None] * NCH
        g[0] = gmake(0)
        g[0].start()
        if NCH > 1:
            g[1] = gmake(1)
            g[1].start()
        for c in range(NCH):
            g[c].wait()
            o[c] = omake(c)
            o[c].start()
            if c + 2 < NCH:
                if c >= 1:
                    o[c - 1].wait()
                g[c + 2] = gmake(c + 2)
                g[c + 2].start()
        for c in range(max(0, NCH - 3), NCH):
            o[c].wait()

    return gather_k


def kernel(tokens):
    B, S, D = tokens.shape
    target = max(_MIN_TOKENS, min(int(S * _SPARSITY_RATIO), _MAX_TOKENS))
    K = min(target, S)

    BS = 1024
    last = S // BS
    idx3, gidx3 = pl.pallas_call(
        functools.partial(_fused_body, S=S, K=K, BS=BS, D=D),
        grid=(B, last),
        in_specs=[pl.BlockSpec((1, BS, D), lambda b, s: (b, s, 0))],
        out_specs=[
            pl.BlockSpec((1, 1, K), lambda b, s: (b, 0, 0)),
            pl.BlockSpec((1, 1, K), lambda b, s: (b, 0, 0)),
        ],
        out_shape=[
            jax.ShapeDtypeStruct((B, 1, K), jnp.int32),
            jax.ShapeDtypeStruct((B, 1, K), jnp.int32),
        ],
        scratch_shapes=[
            pltpu.VMEM((1, S), jnp.float32),
            pltpu.VMEM((S, 1), jnp.float32),
        ],
    )(tokens)

    idx = idx3.reshape(B, K)
    BK = B * K
    bpw = BK // _NW
    CH = 8
    gidx = gidx3.reshape(_NW, bpw // CH, CH)
    table = tokens.reshape(B * S, D)
    out = _make_sc_gather(BK, D, bpw, CH)(table, gidx)
    return out.reshape(B, K, D), idx
